# baseline (speedup 1.0000x reference)
import functools

import jax
import jax.numpy as jnp
import numpy as np
from jax import lax
from jax.experimental import pallas as pl
from jax.experimental.pallas import tpu as pltpu
from jax.experimental.pallas import tpu_sc as plsc

_NOBJ = 50
_NSLOT = 128
_CC = 23
_NC = 21
_CHUNK = _CC * _NSLOT
_SUBS = 8
_LBLK = _CHUNK * _SUBS
_BIGI = 1 << 30


def _sel_matrix():
    m = np.zeros((_CHUNK, 2 * _NSLOT), np.float32)
    li = np.arange(_CHUNK)
    p = li // _CC
    ch = li % _CC
    m[li[ch >= 2], p[ch >= 2]] = 1.0
    m[li[ch == 2], _NSLOT + p[ch == 2]] = 1.0
    return m


def _sc_match(boxes_p, cls_p, prior_p):
    B = boxes_p.shape[0]
    mesh = plsc.VectorSubcoreMesh(core_axis_name="c", subcore_axis_name="s")
    out_type = (
        jax.ShapeDtypeStruct((B, _NSLOT), jnp.int32),
        jax.ShapeDtypeStruct((B, _NSLOT), jnp.float32),
        jax.ShapeDtypeStruct((B, _NSLOT), jnp.float32),
    )
    scratch = [
        pltpu.VMEM((224,), jnp.int32),
        pltpu.VMEM((64,), jnp.int32),
        pltpu.VMEM((208,), jnp.float32),
        pltpu.VMEM((_NSLOT,), jnp.int32),
        pltpu.VMEM((_NSLOT,), jnp.float32),
        pltpu.VMEM((_NSLOT,), jnp.float32),
    ]

    @functools.partial(
        pl.kernel, out_type=out_type, mesh=mesh, scratch_types=scratch,
        compiler_params=pltpu.CompilerParams(needs_layout_passes=False))
    def k(boxes_hbm, cls_hbm, prior_hbm, ocls, oox, ooy,
          box_v, cls_v, prior_v, gcls_v, gox_v, goy_v):
        wid = lax.axis_index("s") * 2 + lax.axis_index("c")

        @pl.when(wid < B)
        def _():
            b = wid
            pltpu.sync_copy(boxes_hbm.at[b], box_v)
            pltpu.sync_copy(cls_hbm.at[b], cls_v)
            pltpu.sync_copy(prior_hbm, prior_v)

            iota = lax.broadcasted_iota(jnp.int32, (16,), 0)
            zero_i = jnp.zeros((16,), jnp.int32)
            zero_f = jnp.zeros((16,), jnp.float32)
            for i in range(_NSLOT // 16):
                gcls_v[pl.ds(i * 16, 16)] = zero_i
                gox_v[pl.ds(i * 16, 16)] = zero_f
                goy_v[pl.ds(i * 16, 16)] = zero_f

            ks, cs, oxs, oys = [], [], [], []
            for c in range(4):
                ridx = jnp.minimum(iota + c * 16, _NOBJ - 1) * 4
                x0 = plsc.load_gather(box_v, [ridx])
                y0 = plsc.load_gather(box_v, [ridx + 1])
                x1 = plsc.load_gather(box_v, [ridx + 2])
                y1 = plsc.load_gather(box_v, [ridx + 3])
                cx = lax.shift_right_arithmetic(x0 + x1, 1)
                cy = lax.shift_right_arithmetic(y0 + y1, 1)
                kc = (lax.shift_right_arithmetic(cy, 5) * 10
                      + lax.shift_right_arithmetic(cx, 5))
                px = plsc.load_gather(prior_v, [kc * 2])
                py = plsc.load_gather(prior_v, [kc * 2 + 1])
                ks.append(kc)
                oxs.append(cx.astype(jnp.float32) - px)
                oys.append(cy.astype(jnp.float32) - py)
                cs.append(cls_v[pl.ds(c * 16, 16)])

            for n in range(_NOBJ):
                c, l = divmod(n, 16)
                m = iota == l
                plsc.store_scatter(gcls_v, [ks[c]], cs[c], mask=m)
                plsc.store_scatter(gox_v, [ks[c]], oxs[c], mask=m)
                plsc.store_scatter(goy_v, [ks[c]], oys[c], mask=m)

            pltpu.sync_copy(gcls_v, ocls.at[b])
            pltpu.sync_copy(gox_v, oox.at[b])
            pltpu.sync_copy(goy_v, ooy.at[b])

    return k(boxes_p, cls_p, prior_p)


def _tc_loss(pred_f, sel, wcls, pox, poy, gcls, gox, goy, interpret=False):
    B, F = pred_f.shape
    P = F // _CC
    NJ = (F + _LBLK - 1) // _LBLK

    def body(pred_ref, sel_ref, wcls_ref, pox_ref, poy_ref,
             cls_ref, ox_ref, oy_ref, loc_ref, conf_ref,
             m1_ref, m2_ref, i1_ref, i2_ref, se0_ref):
        j = pl.program_id(0)
        inf = jnp.float32(jnp.inf)

        @pl.when(j == 0)
        def _():
            m1_ref[...] = jnp.full((B, _NSLOT), inf, jnp.float32)
            m2_ref[...] = jnp.full((B, _NSLOT), inf, jnp.float32)
            i1_ref[...] = jnp.full((B, _NSLOT), _BIGI, jnp.int32)
            i2_ref[...] = jnp.full((B, _NSLOT), _BIGI, jnp.int32)

        x = pred_ref[...]
        fcol = lax.broadcasted_iota(jnp.int32, (1, _LBLK), 1) + j * _LBLK
        x = jnp.where(fcol < F, x, 0.0)
        e = jnp.exp(x)
        lane = lax.broadcasted_iota(jnp.int32, (1, _NSLOT), 1)
        selm = sel_ref[...]
        for k in range(_SUBS):
            es = e[:, k * _CHUNK:(k + 1) * _CHUNK]
            red = jax.lax.dot_general(
                es, selm, (((1,), (0,)), ((), ())),
                preferred_element_type=jnp.float32)
            se = red[:, :_NSLOT]
            e0 = red[:, _NSLOT:]
            pidx = lane + (j * _SUBS + k) * _NSLOT
            r = jnp.where(pidx < P, e0 / se, inf)
            iv = jnp.broadcast_to(pidx, (B, _NSLOT))
            m1v = m1_ref[...]
            m2v = m2_ref[...]
            lt1 = r < m1v
            lt2 = r < m2v
            m2_ref[...] = jnp.where(lt1, m1v, jnp.where(lt2, r, m2v))
            i2_ref[...] = jnp.where(lt1, i1_ref[...],
                                    jnp.where(lt2, iv, i2_ref[...]))
            m1_ref[...] = jnp.where(lt1, r, m1v)
            i1_ref[...] = jnp.where(lt1, iv, i1_ref[...])
            if k == 0:
                @pl.when(j == 0)
                def _():
                    se0_ref[...] = se

        @pl.when(j == NJ - 1)
        def _():
            g = cls_ref[...]
            pos = g > 0
            posf = jnp.where(pos, 1.0, 0.0)
            num_pos = jnp.sum(posf)
            loc_sum = jnp.sum(((ox_ref[...] - pox_ref[...]) ** 2
                               + (oy_ref[...] - poy_ref[...]) ** 2) * posf)

            w = wcls_ref[...]
            rowi = lax.broadcasted_iota(jnp.int32, (1, _NC, 1), 1)
            onehot = jnp.where(rowi == g[:, None, :], 1.0, 0.0)
            picked = jnp.sum(w * onehot, axis=1)
            lse0 = jnp.log(se0_ref[...])
            ce_sum = jnp.sum((lse0 - picked) * posf)

            num_sel = num_pos
            m1a = m1_ref[...]
            m2a = m2_ref[...]
            i1a = i1_ref[...]
            i2a = i2_ref[...]
            for b in range(B):
                a = m1a[b:b + 1]
                ia = i1a[b:b + 1]
                pb = posf[b:b + 1]
                g1 = jnp.min(a)
                s1 = jnp.min(jnp.where(a == g1, ia, _BIGI))
                cond = (ia == s1) & (a == g1)
                a2 = jnp.where(cond, m2a[b:b + 1], a)
                ia2 = jnp.where(cond, i2a[b:b + 1], ia)
                g2 = jnp.min(a2)
                s2 = jnp.min(jnp.where(a2 == g2, ia2, _BIGI))
                isp1 = jnp.sum(jnp.where(lane == s1, pb, 0.0)) > 0.0
                isp2 = jnp.sum(jnp.where(lane == s2, pb, 0.0)) > 0.0
                ce_sum = (ce_sum + jnp.where(isp1, 0.0, -jnp.log(g1))
                          + jnp.where(isp2, 0.0, -jnp.log(g2)))
                num_sel = (num_sel + jnp.where(isp1, 0.0, 1.0)
                           + jnp.where(isp2, 0.0, 1.0))

            loc_ref[0, 0] = loc_sum / (jnp.maximum(num_pos, 1.0) * 2.0)
            conf_ref[0, 0] = ce_sum / jnp.maximum(num_sel, 1.0)

    return pl.pallas_call(
        body,
        grid=(NJ,),
        in_specs=[
            pl.BlockSpec((B, _LBLK), lambda j: (0, j)),
            pl.BlockSpec((_CHUNK, 2 * _NSLOT), lambda j: (0, 0)),
            pl.BlockSpec((B, _NC, _NSLOT), lambda j: (0, 0, 0)),
            pl.BlockSpec((B, _NSLOT), lambda j: (0, 0)),
            pl.BlockSpec((B, _NSLOT), lambda j: (0, 0)),
            pl.BlockSpec((B, _NSLOT), lambda j: (0, 0)),
            pl.BlockSpec((B, _NSLOT), lambda j: (0, 0)),
            pl.BlockSpec((B, _NSLOT), lambda j: (0, 0)),
        ],
        out_specs=[
            pl.BlockSpec(memory_space=pltpu.SMEM),
            pl.BlockSpec(memory_space=pltpu.SMEM),
        ],
        out_shape=[
            jax.ShapeDtypeStruct((1, 1), jnp.float32),
            jax.ShapeDtypeStruct((1, 1), jnp.float32),
        ],
        scratch_shapes=[
            pltpu.VMEM((B, _NSLOT), jnp.float32),
            pltpu.VMEM((B, _NSLOT), jnp.float32),
            pltpu.VMEM((B, _NSLOT), jnp.int32),
            pltpu.VMEM((B, _NSLOT), jnp.int32),
            pltpu.VMEM((B, _NSLOT), jnp.float32),
        ],
        interpret=interpret,
    )(pred_f, sel, wcls, pox, poy, gcls, gox, goy)


def kernel(prediction_3d, boxes, classids, priorBox_2d):
    B, P, _ = prediction_3d.shape
    boxes_p = jnp.pad(boxes.astype(jnp.int32),
                      ((0, 0), (0, 6), (0, 0))).reshape(B, 224)
    cls_p = jnp.pad(classids.astype(jnp.int32), ((0, 0), (0, 14)))
    prior_p = priorBox_2d[:104].reshape(208)
    gcls, gox, goy = _sc_match(boxes_p, cls_p, prior_p)

    pred_f = prediction_3d.reshape(B, P * _CC)
    sel = jnp.asarray(_sel_matrix())
    win = prediction_3d[:, :_NSLOT, :]
    wcls = jnp.transpose(win[:, :, 2:], (0, 2, 1))
    pox = win[:, :, 0]
    poy = win[:, :, 1]
    loc, conf = _tc_loss(pred_f, sel, wcls, pox, poy, gcls, gox, goy)
    return (loc[0, 0], conf[0, 0])


# speedup vs baseline: 1.3678x; 1.3678x over previous
import functools

import jax
import jax.numpy as jnp
import numpy as np
from jax import lax
from jax.experimental import pallas as pl
from jax.experimental.pallas import tpu as pltpu
from jax.experimental.pallas import tpu_sc as plsc

_NOBJ = 50
_NSLOT = 128
_CC = 23
_NC = 21
_PR = 160
_RW = _CC * _PR
_RPB = 20000 // _PR
_RBLK = 200
_BIGI = 1 << 30


def _sel_matrix():
    m = np.zeros((_RW, 2 * _PR), np.float32)
    li = np.arange(_RW)
    p = li // _CC
    ch = li % _CC
    m[li[ch >= 2], p[ch >= 2]] = 1.0
    m[li[ch == 2], _PR + p[ch == 2]] = 1.0
    return m.astype(jnp.bfloat16)


def _sc_match(boxes_p, cls_p, prior_p):
    B = boxes_p.shape[0]
    mesh = plsc.VectorSubcoreMesh(core_axis_name="c", subcore_axis_name="s")
    out_type = (
        jax.ShapeDtypeStruct((B, _NSLOT), jnp.int32),
        jax.ShapeDtypeStruct((B, _NSLOT), jnp.float32),
        jax.ShapeDtypeStruct((B, _NSLOT), jnp.float32),
    )
    scratch = [
        pltpu.VMEM((224,), jnp.int32),
        pltpu.VMEM((64,), jnp.int32),
        pltpu.VMEM((208,), jnp.float32),
        pltpu.VMEM((_NSLOT,), jnp.int32),
        pltpu.VMEM((_NSLOT,), jnp.float32),
        pltpu.VMEM((_NSLOT,), jnp.float32),
    ]

    @functools.partial(
        pl.kernel, out_type=out_type, mesh=mesh, scratch_types=scratch,
        compiler_params=pltpu.CompilerParams(needs_layout_passes=False))
    def k(boxes_hbm, cls_hbm, prior_hbm, ocls, oox, ooy,
          box_v, cls_v, prior_v, gcls_v, gox_v, goy_v):
        wid = lax.axis_index("s") * 2 + lax.axis_index("c")

        @pl.when(wid < B)
        def _():
            b = wid
            pltpu.sync_copy(boxes_hbm.at[b], box_v)
            pltpu.sync_copy(cls_hbm.at[b], cls_v)
            pltpu.sync_copy(prior_hbm, prior_v)

            iota = lax.broadcasted_iota(jnp.int32, (16,), 0)
            zero_i = jnp.zeros((16,), jnp.int32)
            zero_f = jnp.zeros((16,), jnp.float32)
            for i in range(_NSLOT // 16):
                gcls_v[pl.ds(i * 16, 16)] = zero_i
                gox_v[pl.ds(i * 16, 16)] = zero_f
                goy_v[pl.ds(i * 16, 16)] = zero_f

            ks, cs, oxs, oys = [], [], [], []
            for c in range(4):
                ridx = jnp.minimum(iota + c * 16, _NOBJ - 1) * 4
                x0 = plsc.load_gather(box_v, [ridx])
                y0 = plsc.load_gather(box_v, [ridx + 1])
                x1 = plsc.load_gather(box_v, [ridx + 2])
                y1 = plsc.load_gather(box_v, [ridx + 3])
                cx = lax.shift_right_arithmetic(x0 + x1, 1)
                cy = lax.shift_right_arithmetic(y0 + y1, 1)
                kc = (lax.shift_right_arithmetic(cy, 5) * 10
                      + lax.shift_right_arithmetic(cx, 5))
                px = plsc.load_gather(prior_v, [kc * 2])
                py = plsc.load_gather(prior_v, [kc * 2 + 1])
                ks.append(kc)
                oxs.append(cx.astype(jnp.float32) - px)
                oys.append(cy.astype(jnp.float32) - py)
                cs.append(cls_v[pl.ds(c * 16, 16)])

            for n in range(_NOBJ):
                c, l = divmod(n, 16)
                m = iota == l
                plsc.store_scatter(gcls_v, [ks[c]], cs[c], mask=m)
                plsc.store_scatter(gox_v, [ks[c]], oxs[c], mask=m)
                plsc.store_scatter(goy_v, [ks[c]], oys[c], mask=m)

            pltpu.sync_copy(gcls_v, ocls.at[b])
            pltpu.sync_copy(gox_v, oox.at[b])
            pltpu.sync_copy(goy_v, ooy.at[b])

    return k(boxes_p, cls_p, prior_p)


def _lexlt(v1, i1, v2, i2):
    return (v1 < v2) | ((v1 == v2) & (i1 < i2))


def _tc_loss(pred_f, sel, wcls, pox, poy, gcls, gox, goy, interpret=False):
    R, _ = pred_f.shape
    B = R // _RPB
    NJ = R // _RBLK

    def body(pred_ref, sel_ref, wcls_ref, pox_ref, poy_ref,
             cls_ref, ox_ref, oy_ref, loc_ref, conf_ref,
             m1_ref, i1_ref, m2_ref, i2_ref, sew_ref):
        j = pl.program_id(0)
        inf = jnp.inf

        @pl.when(j == 0)
        def _():
            m1_ref[...] = jnp.full((B, _PR), inf, jnp.float32)
            m2_ref[...] = jnp.full((B, _PR), inf, jnp.float32)
            i1_ref[...] = jnp.full((B, _PR), _BIGI, jnp.int32)
            i2_ref[...] = jnp.full((B, _PR), _BIGI, jnp.int32)
            sew_ref[...] = jnp.zeros((B, _PR), jnp.float32)

        x = pred_ref[...]
        e = jnp.exp(x).astype(jnp.bfloat16)
        red = jax.lax.dot_general(
            e, sel_ref[...], (((1,), (0,)), ((), ())),
            preferred_element_type=jnp.float32)
        se = red[:, :_PR]
        e0 = red[:, _PR:]
        r = e0 / se

        trow = (lax.broadcasted_iota(jnp.int32, (_RBLK, 1), 0)
                + j * _RBLK)
        brow = trow // _RPB
        crow = trow - brow * _RPB
        col = lax.broadcasted_iota(jnp.int32, (1, _PR), 1)
        pidx = crow * _PR + col

        rowb = lax.broadcasted_iota(jnp.int32, (B, 1), 0)
        for b in range(B):
            inb = brow == b
            v = jnp.where(inb, r, inf)
            n1 = jnp.min(v, axis=0, keepdims=True)
            j1 = jnp.min(jnp.where(v == n1, pidx, _BIGI),
                         axis=0, keepdims=True)
            v2 = jnp.where(pidx == j1, inf, v)
            n2 = jnp.min(v2, axis=0, keepdims=True)
            j2 = jnp.min(jnp.where(v2 == n2, pidx, _BIGI),
                         axis=0, keepdims=True)
            isb = rowb == b
            M1 = m1_ref[...]
            I1 = i1_ref[...]
            M2 = m2_ref[...]
            I2 = i2_ref[...]
            lt1 = _lexlt(n1, j1, M1, I1)
            w1v = jnp.where(lt1, n1, M1)
            w1i = jnp.where(lt1, j1, I1)
            l1v = jnp.where(lt1, M1, jnp.broadcast_to(n1, M1.shape))
            l1i = jnp.where(lt1, I1, jnp.broadcast_to(j1, I1.shape))
            lt2 = _lexlt(n2, j2, M2, I2)
            c2v = jnp.where(lt2, jnp.broadcast_to(n2, M2.shape), M2)
            c2i = jnp.where(lt2, jnp.broadcast_to(j2, I2.shape), I2)
            lt3 = _lexlt(l1v, l1i, c2v, c2i)
            w2v = jnp.where(lt3, l1v, c2v)
            w2i = jnp.where(lt3, l1i, c2i)
            m1_ref[...] = jnp.where(isb, w1v, M1)
            i1_ref[...] = jnp.where(isb, w1i, I1)
            m2_ref[...] = jnp.where(isb, w2v, M2)
            i2_ref[...] = jnp.where(isb, w2i, I2)
            wrow = jnp.sum(jnp.where(inb & (crow == 0), se, 0.0),
                           axis=0, keepdims=True)
            sew_ref[...] = jnp.where(isb, sew_ref[...] + wrow, sew_ref[...])

        @pl.when(j == NJ - 1)
        def _():
            g = cls_ref[...]
            pos = g > 0
            posf = jnp.where(pos, 1.0, 0.0)
            num_pos = jnp.sum(posf)
            loc_sum = jnp.sum(((ox_ref[...] - pox_ref[...]) ** 2
                               + (oy_ref[...] - poy_ref[...]) ** 2) * posf)

            w = wcls_ref[...]
            ci = lax.broadcasted_iota(jnp.int32, (1, _NC, 1), 1)
            onehot = jnp.where(ci == g[:, None, :], 1.0, 0.0)
            picked = jnp.sum(w * onehot, axis=1)
            lse0 = jnp.log(sew_ref[...][:, :_NSLOT])
            ce_sum = jnp.sum((lse0 - picked) * posf)

            num_sel = num_pos
            lane = lax.broadcasted_iota(jnp.int32, (1, _NSLOT), 1)
            for b in range(B):
                isb = rowb == b
                a = jnp.sum(jnp.where(isb, m1_ref[...], 0.0),
                            axis=0, keepdims=True)
                ia = jnp.sum(jnp.where(isb, i1_ref[...], 0),
                             axis=0, keepdims=True)
                a2r = jnp.sum(jnp.where(isb, m2_ref[...], 0.0),
                              axis=0, keepdims=True)
                ia2r = jnp.sum(jnp.where(isb, i2_ref[...], 0),
                               axis=0, keepdims=True)
                pb = jnp.sum(jnp.where(isb, posf, 0.0),
                             axis=0, keepdims=True)
                g1 = jnp.min(a)
                s1 = jnp.min(jnp.where(a == g1, ia, _BIGI))
                cond = ia == s1
                a2 = jnp.where(cond, a2r, a)
                ia2 = jnp.where(cond, ia2r, ia)
                g2 = jnp.min(a2)
                s2 = jnp.min(jnp.where(a2 == g2, ia2, _BIGI))
                isp1 = jnp.sum(jnp.where(lane == s1, pb, 0.0)) > 0.0
                isp2 = jnp.sum(jnp.where(lane == s2, pb, 0.0)) > 0.0
                ce_sum = (ce_sum + jnp.where(isp1, 0.0, -jnp.log(g1))
                          + jnp.where(isp2, 0.0, -jnp.log(g2)))
                num_sel = (num_sel + jnp.where(isp1, 0.0, 1.0)
                           + jnp.where(isp2, 0.0, 1.0))

            loc_ref[0, 0] = loc_sum / (jnp.maximum(num_pos, 1.0) * 2.0)
            conf_ref[0, 0] = ce_sum / jnp.maximum(num_sel, 1.0)

    return pl.pallas_call(
        body,
        grid=(NJ,),
        in_specs=[
            pl.BlockSpec((_RBLK, _RW), lambda j: (j, 0)),
            pl.BlockSpec((_RW, 2 * _PR), lambda j: (0, 0)),
            pl.BlockSpec((B, _NC, _NSLOT), lambda j: (0, 0, 0)),
            pl.BlockSpec((B, _NSLOT), lambda j: (0, 0)),
            pl.BlockSpec((B, _NSLOT), lambda j: (0, 0)),
            pl.BlockSpec((B, _NSLOT), lambda j: (0, 0)),
            pl.BlockSpec((B, _NSLOT), lambda j: (0, 0)),
            pl.BlockSpec((B, _NSLOT), lambda j: (0, 0)),
        ],
        out_specs=[
            pl.BlockSpec(memory_space=pltpu.SMEM),
            pl.BlockSpec(memory_space=pltpu.SMEM),
        ],
        out_shape=[
            jax.ShapeDtypeStruct((1, 1), jnp.float32),
            jax.ShapeDtypeStruct((1, 1), jnp.float32),
        ],
        scratch_shapes=[
            pltpu.VMEM((8, _PR), jnp.float32),
            pltpu.VMEM((8, _PR), jnp.int32),
            pltpu.VMEM((8, _PR), jnp.float32),
            pltpu.VMEM((8, _PR), jnp.int32),
            pltpu.VMEM((8, _PR), jnp.float32),
        ],
        interpret=interpret,
    )(pred_f, sel, wcls, pox, poy, gcls, gox, goy)


def kernel(prediction_3d, boxes, classids, priorBox_2d):
    B, P, _ = prediction_3d.shape
    boxes_p = jnp.pad(boxes.astype(jnp.int32),
                      ((0, 0), (0, 6), (0, 0))).reshape(B, 224)
    cls_p = jnp.pad(classids.astype(jnp.int32), ((0, 0), (0, 14)))
    prior_p = priorBox_2d[:104].reshape(208)
    gcls, gox, goy = _sc_match(boxes_p, cls_p, prior_p)

    pred_f = prediction_3d.reshape(B * _RPB, _RW)
    sel = _sel_matrix()
    win = prediction_3d[:, :_NSLOT, :]
    wcls = jnp.transpose(win[:, :, 2:], (0, 2, 1))
    pox = win[:, :, 0]
    poy = win[:, :, 1]
    loc, conf = _tc_loss(pred_f, sel, wcls, pox, poy, gcls, gox, goy)
    return (loc[0, 0], conf[0, 0])


# speedup vs baseline: 3.3466x; 2.4467x over previous
import functools

import jax
import jax.numpy as jnp
from jax import lax
from jax.experimental import pallas as pl
from jax.experimental.pallas import tpu as pltpu
from jax.experimental.pallas import tpu_sc as plsc

_NOBJ = 50
_NSLOT = 128


def _sc_match(boxes_p, cls_p, prior_p):
    B = boxes_p.shape[0]
    mesh = plsc.VectorSubcoreMesh(core_axis_name="c", subcore_axis_name="s")
    out_type = (
        jax.ShapeDtypeStruct((B, 1, _NSLOT), jnp.int32),
        jax.ShapeDtypeStruct((B, 1, _NSLOT), jnp.float32),
        jax.ShapeDtypeStruct((B, 1, _NSLOT), jnp.float32),
    )
    scratch = [
        pltpu.VMEM((224,), jnp.int32),
        pltpu.VMEM((64,), jnp.int32),
        pltpu.VMEM((208,), jnp.float32),
        pltpu.VMEM((_NSLOT,), jnp.int32),
        pltpu.VMEM((_NSLOT,), jnp.float32),
        pltpu.VMEM((_NSLOT,), jnp.float32),
    ]

    @functools.partial(
        pl.kernel, out_type=out_type, mesh=mesh, scratch_types=scratch,
        compiler_params=pltpu.CompilerParams(needs_layout_passes=False))
    def k(boxes_hbm, cls_hbm, prior_hbm, ocls, oox, ooy,
          box_v, cls_v, prior_v, gcls_v, gox_v, goy_v):
        wid = lax.axis_index("s") * 2 + lax.axis_index("c")

        @pl.when(wid < B)
        def _():
            b = wid
            pltpu.sync_copy(boxes_hbm.at[b], box_v)
            pltpu.sync_copy(cls_hbm.at[b], cls_v)
            pltpu.sync_copy(prior_hbm, prior_v)

            iota = lax.broadcasted_iota(jnp.int32, (16,), 0)
            zero_i = jnp.zeros((16,), jnp.int32)
            zero_f = jnp.zeros((16,), jnp.float32)
            for i in range(_NSLOT // 16):
                gcls_v[pl.ds(i * 16, 16)] = zero_i
                gox_v[pl.ds(i * 16, 16)] = zero_f
                goy_v[pl.ds(i * 16, 16)] = zero_f

            ks, cs, oxs, oys = [], [], [], []
            for c in range(4):
                ridx = jnp.minimum(iota + c * 16, _NOBJ - 1) * 4
                x0 = plsc.load_gather(box_v, [ridx])
                y0 = plsc.load_gather(box_v, [ridx + 1])
                x1 = plsc.load_gather(box_v, [ridx + 2])
                y1 = plsc.load_gather(box_v, [ridx + 3])
                cx = lax.shift_right_arithmetic(x0 + x1, 1)
                cy = lax.shift_right_arithmetic(y0 + y1, 1)
                kc = (lax.shift_right_arithmetic(cy, 5) * 10
                      + lax.shift_right_arithmetic(cx, 5))
                px = plsc.load_gather(prior_v, [kc * 2])
                py = plsc.load_gather(prior_v, [kc * 2 + 1])
                ks.append(kc)
                oxs.append(cx.astype(jnp.float32) - px)
                oys.append(cy.astype(jnp.float32) - py)
                cs.append(cls_v[pl.ds(c * 16, 16)])

            for n in range(_NOBJ):
                c, l = divmod(n, 16)
                m = iota == l
                plsc.store_scatter(gcls_v, [ks[c]], cs[c], mask=m)
                plsc.store_scatter(gox_v, [ks[c]], oxs[c], mask=m)
                plsc.store_scatter(goy_v, [ks[c]], oys[c], mask=m)

            pltpu.sync_copy(gcls_v, ocls.at[b, 0])
            pltpu.sync_copy(gox_v, oox.at[b, 0])
            pltpu.sync_copy(goy_v, ooy.at[b, 0])

    return k(boxes_p, cls_p, prior_p)


def _tc_loss(pred_r, pox, poy, gcls, gox, goy, interpret=False):
    B, NC, P = pred_r.shape

    def body(pred_ref, pox_ref, poy_ref, cls_ref, ox_ref, oy_ref,
             loc_ref, conf_ref, acc_ref):
        b = pl.program_id(0)
        x = pred_ref[0].astype(jnp.float32)
        e = jnp.exp(x)
        se = jax.lax.dot_general(
            jnp.ones((1, NC), jnp.float32), e, (((1,), (0,)), ((), ())),
            preferred_element_type=jnp.float32)
        r = e[0:1, :] / se

        col = lax.broadcasted_iota(jnp.int32, (1, P), 1)
        m1 = jnp.min(r)
        i1 = jnp.min(jnp.where(r == m1, col, P))
        r2 = jnp.where(col == i1, jnp.float32(jnp.inf), r)
        m2 = jnp.min(r2)
        i2 = jnp.min(jnp.where(r2 == m2, col, P))

        g2 = cls_ref[0]
        pos = g2 > 0
        posf = jnp.where(pos, 1.0, 0.0)
        npos_b = jnp.sum(posf)

        loc_b = jnp.sum(((ox_ref[0] - pox_ref[0]) ** 2
                         + (oy_ref[0] - poy_ref[0]) ** 2) * posf)

        xs = x[:, 0:_NSLOT]
        rowi = lax.broadcasted_iota(jnp.int32, (NC, 1), 0)
        onehot = jnp.where(rowi == g2, 1.0, 0.0)
        picked = jnp.sum(xs * onehot, axis=0, keepdims=True)
        lse_s = jnp.log(se[:, 0:_NSLOT])
        ce_b = jnp.sum((lse_s - picked) * posf)

        lane = lax.broadcasted_iota(jnp.int32, (1, _NSLOT), 1)
        isp1 = jnp.sum(jnp.where((lane == i1) & pos, 1.0, 0.0)) > 0.0
        isp2 = jnp.sum(jnp.where((lane == i2) & pos, 1.0, 0.0)) > 0.0
        ce_b = (ce_b + jnp.where(isp1, 0.0, -jnp.log(m1))
                + jnp.where(isp2, 0.0, -jnp.log(m2)))
        nsel_b = npos_b + jnp.where(isp1, 0.0, 1.0) + jnp.where(isp2, 0.0, 1.0)

        @pl.when(b == 0)
        def _():
            acc_ref[0] = loc_b
            acc_ref[1] = npos_b
            acc_ref[2] = ce_b
            acc_ref[3] = nsel_b

        @pl.when(b > 0)
        def _():
            acc_ref[0] += loc_b
            acc_ref[1] += npos_b
            acc_ref[2] += ce_b
            acc_ref[3] += nsel_b

        loc_ref[0, 0] = acc_ref[0] / (jnp.maximum(acc_ref[1], 1.0) * 2.0)
        conf_ref[0, 0] = acc_ref[2] / jnp.maximum(acc_ref[3], 1.0)

    return pl.pallas_call(
        body,
        grid=(B,),
        in_specs=[
            pl.BlockSpec((1, NC, P), lambda b: (b, 0, 0)),
            pl.BlockSpec((1, 1, _NSLOT), lambda b: (b, 0, 0)),
            pl.BlockSpec((1, 1, _NSLOT), lambda b: (b, 0, 0)),
            pl.BlockSpec((1, 1, _NSLOT), lambda b: (b, 0, 0)),
            pl.BlockSpec((1, 1, _NSLOT), lambda b: (b, 0, 0)),
            pl.BlockSpec((1, 1, _NSLOT), lambda b: (b, 0, 0)),
        ],
        out_specs=[
            pl.BlockSpec(memory_space=pltpu.SMEM),
            pl.BlockSpec(memory_space=pltpu.SMEM),
        ],
        out_shape=[
            jax.ShapeDtypeStruct((1, 1), jnp.float32),
            jax.ShapeDtypeStruct((1, 1), jnp.float32),
        ],
        scratch_shapes=[pltpu.SMEM((4,), jnp.float32)],
        interpret=interpret,
    )(pred_r, pox, poy, gcls, gox, goy)


def kernel(prediction_3d, boxes, classids, priorBox_2d):
    B = prediction_3d.shape[0]
    boxes_p = jnp.pad(boxes.astype(jnp.int32),
                      ((0, 0), (0, 6), (0, 0))).reshape(B, 224)
    cls_p = jnp.pad(classids.astype(jnp.int32), ((0, 0), (0, 14)))
    prior_p = priorBox_2d[:104].reshape(208)
    gcls, gox, goy = _sc_match(boxes_p, cls_p, prior_p)
    pred_r = jnp.transpose(prediction_3d[:, :, 2:],
                           (0, 2, 1)).astype(jnp.bfloat16)
    po = prediction_3d[:, :_NSLOT, :2]
    pox = po[:, :, 0].reshape(B, 1, _NSLOT)
    poy = po[:, :, 1].reshape(B, 1, _NSLOT)
    loc, conf = _tc_loss(pred_r, pox, poy, gcls, gox, goy)
    return (loc[0, 0], conf[0, 0])


# speedup vs baseline: 3.5360x; 1.0566x over previous
import functools

import jax
import jax.numpy as jnp
from jax import lax
from jax.experimental import pallas as pl
from jax.experimental.pallas import tpu as pltpu
from jax.experimental.pallas import tpu_sc as plsc

_NOBJ = 50
_NSLOT = 128


def _sc_match(boxes_p, cls_p, prior_p):
    B = boxes_p.shape[0]
    mesh = plsc.VectorSubcoreMesh(core_axis_name="c", subcore_axis_name="s")
    out_type = (
        jax.ShapeDtypeStruct((B, 1, _NSLOT), jnp.int32),
        jax.ShapeDtypeStruct((B, 1, _NSLOT), jnp.float32),
        jax.ShapeDtypeStruct((B, 1, _NSLOT), jnp.float32),
    )
    scratch = [
        pltpu.VMEM((224,), jnp.int32),
        pltpu.VMEM((64,), jnp.int32),
        pltpu.VMEM((208,), jnp.float32),
        pltpu.VMEM((_NSLOT,), jnp.int32),
        pltpu.VMEM((_NSLOT,), jnp.float32),
        pltpu.VMEM((_NSLOT,), jnp.float32),
    ]

    @functools.partial(
        pl.kernel, out_type=out_type, mesh=mesh, scratch_types=scratch,
        compiler_params=pltpu.CompilerParams(needs_layout_passes=False))
    def k(boxes_hbm, cls_hbm, prior_hbm, ocls, oox, ooy,
          box_v, cls_v, prior_v, gcls_v, gox_v, goy_v):
        wid = lax.axis_index("s") * 2 + lax.axis_index("c")

        @pl.when(wid < B)
        def _():
            b = wid
            pltpu.sync_copy(boxes_hbm.at[b], box_v)
            pltpu.sync_copy(cls_hbm.at[b], cls_v)
            pltpu.sync_copy(prior_hbm, prior_v)

            iota = lax.broadcasted_iota(jnp.int32, (16,), 0)
            zero_i = jnp.zeros((16,), jnp.int32)
            zero_f = jnp.zeros((16,), jnp.float32)
            for i in range(_NSLOT // 16):
                gcls_v[pl.ds(i * 16, 16)] = zero_i
                gox_v[pl.ds(i * 16, 16)] = zero_f
                goy_v[pl.ds(i * 16, 16)] = zero_f

            ks, cs, oxs, oys = [], [], [], []
            for c in range(4):
                ridx = jnp.minimum(iota + c * 16, _NOBJ - 1) * 4
                x0 = plsc.load_gather(box_v, [ridx])
                y0 = plsc.load_gather(box_v, [ridx + 1])
                x1 = plsc.load_gather(box_v, [ridx + 2])
                y1 = plsc.load_gather(box_v, [ridx + 3])
                cx = lax.shift_right_arithmetic(x0 + x1, 1)
                cy = lax.shift_right_arithmetic(y0 + y1, 1)
                kc = (lax.shift_right_arithmetic(cy, 5) * 10
                      + lax.shift_right_arithmetic(cx, 5))
                px = plsc.load_gather(prior_v, [kc * 2])
                py = plsc.load_gather(prior_v, [kc * 2 + 1])
                ks.append(kc)
                oxs.append(cx.astype(jnp.float32) - px)
                oys.append(cy.astype(jnp.float32) - py)
                cs.append(cls_v[pl.ds(c * 16, 16)])

            for n in range(_NOBJ):
                c, l = divmod(n, 16)
                m = iota == l
                plsc.store_scatter(gcls_v, [ks[c]], cs[c], mask=m)
                plsc.store_scatter(gox_v, [ks[c]], oxs[c], mask=m)
                plsc.store_scatter(goy_v, [ks[c]], oys[c], mask=m)

            pltpu.sync_copy(gcls_v, ocls.at[b, 0])
            pltpu.sync_copy(gox_v, oox.at[b, 0])
            pltpu.sync_copy(goy_v, ooy.at[b, 0])

    return k(boxes_p, cls_p, prior_p)


def _tc_loss(pred_r, pox, poy, gcls, gox, goy, interpret=False):
    B, NC, P = pred_r.shape

    def body(pred_ref, pox_ref, poy_ref, cls_ref, ox_ref, oy_ref,
             loc_ref, conf_ref, acc_ref):
        b = pl.program_id(0)
        x = pred_ref[0]
        e = jnp.exp(x)
        se = jax.lax.dot_general(
            jnp.ones((1, NC), jnp.float32), e, (((1,), (0,)), ((), ())),
            preferred_element_type=jnp.float32)
        r = e[0:1, :] / se

        col = lax.broadcasted_iota(jnp.int32, (1, P), 1)
        m1 = jnp.min(r)
        i1 = jnp.min(jnp.where(r == m1, col, P))
        r2 = jnp.where(col == i1, jnp.float32(jnp.inf), r)
        m2 = jnp.min(r2)
        i2 = jnp.min(jnp.where(r2 == m2, col, P))

        g2 = cls_ref[0]
        pos = g2 > 0
        posf = jnp.where(pos, 1.0, 0.0)
        npos_b = jnp.sum(posf)

        loc_b = jnp.sum(((ox_ref[0] - pox_ref[0]) ** 2
                         + (oy_ref[0] - poy_ref[0]) ** 2) * posf)

        xs = x[:, 0:_NSLOT]
        rowi = lax.broadcasted_iota(jnp.int32, (NC, 1), 0)
        onehot = jnp.where(rowi == g2, 1.0, 0.0)
        picked = jnp.sum(xs * onehot, axis=0, keepdims=True)
        lse_s = jnp.log(se[:, 0:_NSLOT])
        ce_b = jnp.sum((lse_s - picked) * posf)

        lane = lax.broadcasted_iota(jnp.int32, (1, _NSLOT), 1)
        isp1 = jnp.sum(jnp.where((lane == i1) & pos, 1.0, 0.0)) > 0.0
        isp2 = jnp.sum(jnp.where((lane == i2) & pos, 1.0, 0.0)) > 0.0
        ce_b = (ce_b + jnp.where(isp1, 0.0, -jnp.log(m1))
                + jnp.where(isp2, 0.0, -jnp.log(m2)))
        nsel_b = npos_b + jnp.where(isp1, 0.0, 1.0) + jnp.where(isp2, 0.0, 1.0)

        @pl.when(b == 0)
        def _():
            acc_ref[0] = loc_b
            acc_ref[1] = npos_b
            acc_ref[2] = ce_b
            acc_ref[3] = nsel_b

        @pl.when(b > 0)
        def _():
            acc_ref[0] += loc_b
            acc_ref[1] += npos_b
            acc_ref[2] += ce_b
            acc_ref[3] += nsel_b

        loc_ref[0, 0] = acc_ref[0] / (jnp.maximum(acc_ref[1], 1.0) * 2.0)
        conf_ref[0, 0] = acc_ref[2] / jnp.maximum(acc_ref[3], 1.0)

    return pl.pallas_call(
        body,
        grid=(B,),
        in_specs=[
            pl.BlockSpec((1, NC, P), lambda b: (b, 0, 0)),
            pl.BlockSpec((1, 1, _NSLOT), lambda b: (b, 0, 0)),
            pl.BlockSpec((1, 1, _NSLOT), lambda b: (b, 0, 0)),
            pl.BlockSpec((1, 1, _NSLOT), lambda b: (b, 0, 0)),
            pl.BlockSpec((1, 1, _NSLOT), lambda b: (b, 0, 0)),
            pl.BlockSpec((1, 1, _NSLOT), lambda b: (b, 0, 0)),
        ],
        out_specs=[
            pl.BlockSpec(memory_space=pltpu.SMEM),
            pl.BlockSpec(memory_space=pltpu.SMEM),
        ],
        out_shape=[
            jax.ShapeDtypeStruct((1, 1), jnp.float32),
            jax.ShapeDtypeStruct((1, 1), jnp.float32),
        ],
        scratch_shapes=[pltpu.SMEM((4,), jnp.float32)],
        interpret=interpret,
    )(pred_r, pox, poy, gcls, gox, goy)


def kernel(prediction_3d, boxes, classids, priorBox_2d):
    B = prediction_3d.shape[0]
    boxes_p = jnp.pad(boxes.astype(jnp.int32),
                      ((0, 0), (0, 6), (0, 0))).reshape(B, 224)
    cls_p = jnp.pad(classids.astype(jnp.int32), ((0, 0), (0, 14)))
    prior_p = priorBox_2d[:104].reshape(208)
    gcls, gox, goy = _sc_match(boxes_p, cls_p, prior_p)
    pred_r = jnp.transpose(prediction_3d[:, :, 2:], (0, 2, 1))
    po = prediction_3d[:, :_NSLOT, :2]
    pox = po[:, :, 0].reshape(B, 1, _NSLOT)
    poy = po[:, :, 1].reshape(B, 1, _NSLOT)
    loc, conf = _tc_loss(pred_r, pox, poy, gcls, gox, goy)
    return (loc[0, 0], conf[0, 0])
